# baseline (speedup 1.0000x reference)
import functools

import jax
import jax.numpy as jnp
from jax import lax
from jax.experimental import pallas as pl
from jax.experimental.pallas import tpu as pltpu
from jax.experimental.pallas import tpu_sc as plsc

_T, _N, _E, _F = 3, 10000, 160000, 128
_NP = 10240
_NPX = _NP + _F
_NB = _NP // _F
_TE = _T * _E
_ECH = _E // _F
_GCH = _TE // _F
_NW = 32
_NEG = -3e38

_SS_WIN = 48
_SS_SPAN = 384
_EG_CPW = -(-_GCH // _NW)
_TEP = _NW * _EG_CPW * _F


def _summ_gru(Hs, rows_ref, p_ref, nrm_ref, Wt_ref, gru_refs,
              S_scr, idx_smem, yd_smem):
    WZt, UZt, BZt, WRt, URt, BRt, WHt, UHt, BHt = gru_refs
    dot = functools.partial(jnp.dot, preferred_element_type=jnp.float32)
    ydot = dot(Hs, p_ref[...])
    y2d = ydot.reshape(_NB, _F)
    flatidx = (lax.broadcasted_iota(jnp.int32, (_NB, _F), 0) * _F
               + lax.broadcasted_iota(jnp.int32, (_NB, _F), 1))
    y2d = jnp.where(flatidx < _N, y2d, _NEG)

    def tk(i, y):
        m = jnp.max(y)
        am = jnp.min(jnp.where(y >= m, flatidx, jnp.int32(2 ** 30)))
        idx_smem[i] = am
        yd_smem[i] = m
        return jnp.where(flatidx == am, _NEG, y)

    lax.fori_loop(0, _F, tk, y2d)

    nrm = nrm_ref[0, 0]

    def gather(i, _):
        si = idx_smem[i]
        yv = yd_smem[i] / nrm
        S_scr[pl.ds(i, 1), :] = rows_ref[pl.ds(si, 1), :] * yv
        return 0

    lax.fori_loop(0, _F, gather, 0)

    S = S_scr[...]
    Wt = Wt_ref[...]
    Zt = jax.nn.sigmoid(dot(S, WZt[...]) + dot(Wt, UZt[...]) + BZt[...])
    Rt = jax.nn.sigmoid(dot(S, WRt[...]) + dot(Wt, URt[...]) + BRt[...])
    Htt = jnp.tanh(dot(S, WHt[...]) + dot(Rt * Wt, UHt[...]) + BHt[...])
    return (1.0 - Zt) * Wt + Zt * Htt


def _stageA_body(X_ref, p_ref, nrm_ref, Wt_ref,
                 WZt, UZt, BZt, WRt, URt, BRt, WHt, UHt, BHt,
                 Wt_out, S_scr, idx_smem, yd_smem):
    Hs = X_ref[...]
    Wt_out[...] = _summ_gru(Hs, X_ref, p_ref, nrm_ref, Wt_ref,
                            (WZt, UZt, BZt, WRt, URt, BRt, WHt, UHt, BHt),
                            S_scr, idx_smem, yd_smem)


def _tc_stageA(Xp_t, p_col, nrm, Wt, gru_w):
    return pl.pallas_call(
        _stageA_body,
        out_shape=jax.ShapeDtypeStruct((_F, _F), jnp.float32),
        scratch_shapes=[pltpu.VMEM((_F, _F), jnp.float32),
                        pltpu.SMEM((_F,), jnp.int32),
                        pltpu.SMEM((_F,), jnp.float32)],
    )(Xp_t, p_col, nrm, Wt, *gru_w)


def _stageB_body(AH_ref, WtA_ref, p_ref, nrm_ref, Wt_ref,
                 WZt, UZt, BZt, WRt, URt, BRt, WHt, UHt, BHt,
                 Wt_out, Hs_out, S_scr, idx_smem, yd_smem):
    AH = AH_ref[pl.ds(0, _NP), :]
    Wn = WtA_ref[...].T
    Hs = jnp.maximum(jnp.dot(AH, Wn, preferred_element_type=jnp.float32), 0.0)
    Hs_out[...] = Hs
    Wt_out[...] = _summ_gru(Hs, Hs_out, p_ref, nrm_ref, Wt_ref,
                            (WZt, UZt, BZt, WRt, URt, BRt, WHt, UHt, BHt),
                            S_scr, idx_smem, yd_smem)


def _tc_stageB(AH, WtA, p_col, nrm, Wt, gru_w):
    return pl.pallas_call(
        _stageB_body,
        out_shape=[jax.ShapeDtypeStruct((_F, _F), jnp.float32),
                   jax.ShapeDtypeStruct((_NP, _F), jnp.float32)],
        scratch_shapes=[pltpu.VMEM((_F, _F), jnp.float32),
                        pltpu.SMEM((_F,), jnp.int32),
                        pltpu.SMEM((_F,), jnp.float32)],
    )(AH, WtA, p_col, nrm, Wt, *gru_w)


def _stageC_body(AH_ref, W2t_ref, Ut_ref, Ub_ref, P_out, Q_out):
    AH = AH_ref[pl.ds(0, _NP), :]
    W2n = W2t_ref[...].T
    Y = jnp.dot(AH, W2n, preferred_element_type=jnp.float32)
    P_out[...] = jnp.dot(Y, Ut_ref[...], preferred_element_type=jnp.float32)
    Q_out[...] = jnp.dot(Y, Ub_ref[...], preferred_element_type=jnp.float32)


def _tc_stageC(AH, W2t, Ut, Ub):
    return pl.pallas_call(
        _stageC_body,
        out_shape=[jax.ShapeDtypeStruct((_NP, _F), jnp.float32),
                   jax.ShapeDtypeStruct((_NP, _F), jnp.float32)],
    )(AH, W2t, Ut, Ub)


_sc_mesh = plsc.VectorSubcoreMesh(core_axis_name="c", subcore_axis_name="s")


@functools.partial(
    pl.kernel, mesh=_sc_mesh,
    out_type=jax.ShapeDtypeStruct((_NPX, _F), jnp.float32),
    scratch_types=[
        pltpu.VMEM((_SS_WIN, _F), jnp.int32),
        pltpu.VMEM((_F, _F), jnp.float32),
        pltpu.VMEM((_SS_SPAN, _F), jnp.float32),
        pltpu.VMEM((_SS_SPAN // _F, _F), jnp.int32),
        pltpu.VMEM((1, 16), jnp.int32),
        pltpu.VMEM((_SS_WIN, _F), jnp.float32),
        pltpu.VMEM((1, _F), jnp.float32),
        pltpu.VMEM((_SS_WIN, _F), jnp.int32),
        pltpu.VMEM((16,), jnp.int32),
        pltpu.VMEM((1, _F), jnp.int32),
        pltpu.SemaphoreType.DMA,
    ])
def _csr_segsum(H_hbm, ss_hbm, sv_hbm, st_hbm, bounds_hbm, out_hbm,
                ssv, rowb, outb, idxb, bndv, svv, accv, stv, prevv, colv,
                sem):
    c = lax.axis_index("c")
    s = lax.axis_index("s")
    w = s * 2 + c

    pltpu.sync_copy(bounds_hbm.at[w], bndv)
    b = bndv[0, :]
    a0 = pl.multiple_of(b[0], 8)
    n0 = b[3]
    n1 = b[4]

    pltpu.sync_copy(ss_hbm.at[pl.ds(a0, _SS_WIN)], ssv)
    pltpu.sync_copy(sv_hbm.at[pl.ds(a0, _SS_WIN)], svv)
    pltpu.sync_copy(st_hbm.at[pl.ds(a0, _SS_WIN)], stv)

    zero16 = jnp.zeros((16,), jnp.float32)
    zidx16 = jnp.zeros((16,), jnp.int32)
    i16 = lax.broadcasted_iota(jnp.int32, (16,), 0)

    def zo(r, _):
        for cc in range(8):
            outb[r, pl.ds(cc * 16, 16)] = zero16
        return 0
    lax.fori_loop(0, _SS_SPAN, zo, 0)

    def flush(prevs):
        ro = prevs - n0
        for cc in range(8):
            outb[ro, pl.ds(cc * 16, 16)] = accv[0, pl.ds(cc * 16, 16)]

    def lane(l, prevs, bb, vv, tv):
        r = bb * 16 + l
        v16 = vv.at[zidx16 + l].get(mode="promise_in_bounds")
        stc16 = tv.at[zidx16 + l].get(mode="promise_in_bounds")
        prevv[...] = stc16 + zidx16
        stcs = prevv[...][0]
        change = stcs != prevs

        @pl.when(change & (prevs >= n0) & (prevs < n1))
        def _():
            flush(prevs)

        for cc in range(8):
            accv[0, pl.ds(cc * 16, 16)] = (
                rowb[r, pl.ds(cc * 16, 16)] * v16
                + jnp.where(change, zero16, accv[0, pl.ds(cc * 16, 16)]))
        return stcs

    def chunk(kc, prevs):
        pltpu.async_copy(H_hbm.at[ssv.at[kc]], rowb, sem).wait()
        for bb in range(8):
            vv = svv[kc, pl.ds(bb * 16, 16)]
            tv = stv[kc, pl.ds(bb * 16, 16)]
            prevs = lax.fori_loop(
                0, 16, functools.partial(lane, bb=bb, vv=vv, tv=tv), prevs)
        return prevs

    prevf = lax.fori_loop(0, _SS_WIN, chunk, jnp.int32(-1))

    @pl.when((prevf >= n0) & (prevf < n1))
    def _():
        flush(prevf)

    l16 = lax.broadcasted_iota(jnp.int32, (16,), 0)
    nrow = n1 - n0
    for j in range(_SS_SPAN // _F):
        for g in range(8):
            rid = j * _F + g * 16 + l16
            idxb[j, pl.ds(g * 16, 16)] = jnp.where(
                rid < nrow, n0 + rid, _NP + (rid % _F))
    for j in range(_SS_SPAN // _F):
        pltpu.sync_copy(outb.at[pl.ds(j * _F, _F)], out_hbm.at[idxb.at[j]])


@functools.partial(
    pl.kernel, mesh=_sc_mesh,
    out_type=jax.ShapeDtypeStruct((_TEP, _F), jnp.float32),
    scratch_types=[
        pltpu.VMEM((_EG_CPW, _F), jnp.int32),
        pltpu.VMEM((_EG_CPW, _F), jnp.int32),
        pltpu.VMEM((_F, _F), jnp.float32),
        pltpu.VMEM((_F, _F), jnp.float32),
        pltpu.SemaphoreType.DMA,
        pltpu.SemaphoreType.DMA,
    ])
def _edge_gather(P_hbm, Q_hbm, sidx_hbm, tidx_hbm, out_hbm,
                 sv, tv, bufa, bufb, sema, semb):
    c = lax.axis_index("c")
    s = lax.axis_index("s")
    w = s * 2 + c
    base = w * _EG_CPW

    pltpu.sync_copy(sidx_hbm.at[w], sv)
    pltpu.sync_copy(tidx_hbm.at[w], tv)

    def chunk(i, _):
        cpa = pltpu.async_copy(P_hbm.at[sv.at[i]], bufa, sema)
        cpb = pltpu.async_copy(Q_hbm.at[tv.at[i]], bufb, semb)
        cpa.wait()
        cpb.wait()

        def addrow(r, _):
            for cc in range(8):
                bufa[r, pl.ds(cc * 16, 16)] = (
                    bufa[r, pl.ds(cc * 16, 16)]
                    + bufb[r, pl.ds(cc * 16, 16)])
            return 0

        lax.fori_loop(0, _F, addrow, 0)
        pltpu.sync_copy(bufa, out_hbm.at[pl.ds((base + i) * _F, _F)])
        return 0

    lax.fori_loop(0, _EG_CPW, chunk, 0)


def kernel(X, edges, A_values, W_init, W_init2, p,
           W_Z, U_Z, B_Z, W_R, U_R, B_R, W_H, U_H, B_H,
           p2, W_Z2, U_Z2, B_Z2, W_R2, U_R2, B_R2, W_H2, U_H2, B_H2, U):
    f32 = jnp.float32
    i32 = jnp.int32
    edges = edges.astype(i32)
    Xp = jnp.pad(X.astype(f32), ((0, 0), (0, _NP - _N), (0, 0)))

    src2 = edges[1].reshape(_T, _E)
    trg2 = edges[2].reshape(_T, _E)
    vals2 = A_values.astype(f32)

    st3, ss3, sv3 = lax.sort((trg2, src2, vals2), num_keys=1, is_stable=True)

    epadlen = _ECH * _F + (_SS_WIN + 8) * _F
    epad = epadlen - _E
    ssp = jnp.pad(ss3, ((0, 0), (0, epad))).reshape(_T, -1, _F)
    svp = jnp.pad(sv3, ((0, 0), (0, epad))).reshape(_T, -1, _F)
    stp = jnp.pad(st3, ((0, 0), (0, epad)),
                  constant_values=_NPX).reshape(_T, -1, _F)

    probe = jnp.arange(1, _NW, dtype=i32) * (_E // _NW)
    nodes_at = jnp.take_along_axis(st3, probe[None, :].repeat(_T, 0), axis=1)
    Bmid = jax.vmap(lambda a, v: jnp.searchsorted(a, v, side="left"))(
        st3, nodes_at).astype(i32)
    zc = jnp.zeros((_T, 1), i32)
    B = jnp.concatenate([zc, Bmid, jnp.full((_T, 1), _E, i32)], axis=1)
    nmid = nodes_at
    nvec = jnp.concatenate([zc, nmid, jnp.full((_T, 1), _NP, i32)], axis=1)
    a0 = (B[:, :_NW] // (8 * _F)) * 8
    bounds = jnp.stack([
        a0,
        B[:, :_NW] - a0 * _F,
        B[:, 1:] - a0 * _F,
        nvec[:, :_NW],
        nvec[:, 1:],
    ], axis=-1)
    bounds = jnp.pad(bounds, ((0, 0), (0, 0), (0, 11)))
    bounds = bounds.reshape(_T, _NW, 1, 16)

    gpad = _TEP - _TE
    srcf3d = jnp.pad(edges[0] * _NP + edges[1], (0, gpad)
                     ).reshape(_NW, _EG_CPW, _F)
    trgf3d = jnp.pad(edges[0] * _NP + edges[2], (0, gpad)
                     ).reshape(_NW, _EG_CPW, _F)

    gru1 = tuple(m.T for m in (W_Z, U_Z, B_Z, W_R, U_R, B_R, W_H, U_H, B_H))
    gru2 = tuple(m.T for m in (W_Z2, U_Z2, B_Z2, W_R2, U_R2, B_R2,
                               W_H2, U_H2, B_H2))
    p_col = p.reshape(_F, 1)
    p2_col = p2.reshape(_F, 1)
    nrm1 = jnp.linalg.norm(p).reshape(1, 1)
    nrm2 = jnp.linalg.norm(p2).reshape(1, 1)
    Ut = U[:_F]
    Ub = U[_F:]

    Wt = W_init.T
    W2t = W_init2.T
    Ps, Qs = [], []
    for t in range(_T):
        Wt = _tc_stageA(Xp[t], p_col, nrm1, Wt, gru1)
        AH1 = _csr_segsum(Xp[t], ssp[t], svp[t], stp[t], bounds[t])
        W2t, Xs1 = _tc_stageB(AH1, Wt, p2_col, nrm2, W2t, gru2)
        AH2 = _csr_segsum(Xs1, ssp[t], svp[t], stp[t], bounds[t])
        P_t, Q_t = _tc_stageC(AH2, W2t, Ut, Ub)
        Ps.append(P_t)
        Qs.append(Q_t)

    Pfull = jnp.concatenate(Ps, axis=0)
    Qfull = jnp.concatenate(Qs, axis=0)
    out = _edge_gather(Pfull, Qfull, srcf3d, trgf3d)
    return (out[:_TE], Wt.T, W2t.T)


# speedup vs baseline: 1.1284x; 1.1284x over previous
import functools

import jax
import jax.numpy as jnp
from jax import lax
from jax.experimental import pallas as pl
from jax.experimental.pallas import tpu as pltpu
from jax.experimental.pallas import tpu_sc as plsc

_T, _N, _E, _F = 3, 10000, 160000, 128
_NP = 10240
_NPX = _NP + _F
_NB = _NP // _F
_TE = _T * _E
_ECH = _E // _F
_GCH = _TE // _F
_NW = 32
_NEG = -3e38

_SS_WIN = 48
_SS_SPAN = 384
_EG_CPW = -(-_GCH // _NW)
_TEP = _NW * _EG_CPW * _F


def _summ_gru(Hs, rows_ref, p_ref, nrm_ref, Wt_ref, gru_refs,
              S_scr, idx_smem, yd_smem):
    WZt, UZt, BZt, WRt, URt, BRt, WHt, UHt, BHt = gru_refs
    dot = functools.partial(jnp.dot, preferred_element_type=jnp.float32)
    ydot = dot(Hs, p_ref[...])
    y2d = ydot.reshape(_NB, _F)
    flatidx = (lax.broadcasted_iota(jnp.int32, (_NB, _F), 0) * _F
               + lax.broadcasted_iota(jnp.int32, (_NB, _F), 1))
    y2d = jnp.where(flatidx < _N, y2d, _NEG)

    def tk(i, y):
        m = jnp.max(y)
        am = jnp.min(jnp.where(y >= m, flatidx, jnp.int32(2 ** 30)))
        idx_smem[i] = am
        yd_smem[i] = m
        return jnp.where(flatidx == am, _NEG, y)

    lax.fori_loop(0, _F, tk, y2d)

    nrm = nrm_ref[0, 0]

    def gather(i, _):
        si = idx_smem[i]
        yv = yd_smem[i] / nrm
        S_scr[pl.ds(i, 1), :] = rows_ref[pl.ds(si, 1), :] * yv
        return 0

    lax.fori_loop(0, _F, gather, 0)

    S = S_scr[...]
    Wt = Wt_ref[...]
    Zt = jax.nn.sigmoid(dot(S, WZt[...]) + dot(Wt, UZt[...]) + BZt[...])
    Rt = jax.nn.sigmoid(dot(S, WRt[...]) + dot(Wt, URt[...]) + BRt[...])
    Htt = jnp.tanh(dot(S, WHt[...]) + dot(Rt * Wt, UHt[...]) + BHt[...])
    return (1.0 - Zt) * Wt + Zt * Htt


def _stageA_body(X_ref, p_ref, nrm_ref, Wt_ref,
                 WZt, UZt, BZt, WRt, URt, BRt, WHt, UHt, BHt,
                 Wt_out, S_scr, idx_smem, yd_smem):
    Hs = X_ref[...]
    Wt_out[...] = _summ_gru(Hs, X_ref, p_ref, nrm_ref, Wt_ref,
                            (WZt, UZt, BZt, WRt, URt, BRt, WHt, UHt, BHt),
                            S_scr, idx_smem, yd_smem)


def _tc_stageA(Xp_t, p_col, nrm, Wt, gru_w):
    return pl.pallas_call(
        _stageA_body,
        out_shape=jax.ShapeDtypeStruct((_F, _F), jnp.float32),
        scratch_shapes=[pltpu.VMEM((_F, _F), jnp.float32),
                        pltpu.SMEM((_F,), jnp.int32),
                        pltpu.SMEM((_F,), jnp.float32)],
    )(Xp_t, p_col, nrm, Wt, *gru_w)


def _stageB_body(AH_ref, WtA_ref, p_ref, nrm_ref, Wt_ref,
                 WZt, UZt, BZt, WRt, URt, BRt, WHt, UHt, BHt,
                 Wt_out, Hs_out, S_scr, idx_smem, yd_smem):
    AH = AH_ref[pl.ds(0, _NP), :]
    Wn = WtA_ref[...].T
    Hs = jnp.maximum(jnp.dot(AH, Wn, preferred_element_type=jnp.float32), 0.0)
    Hs_out[...] = Hs
    Wt_out[...] = _summ_gru(Hs, Hs_out, p_ref, nrm_ref, Wt_ref,
                            (WZt, UZt, BZt, WRt, URt, BRt, WHt, UHt, BHt),
                            S_scr, idx_smem, yd_smem)


def _tc_stageB(AH, WtA, p_col, nrm, Wt, gru_w):
    return pl.pallas_call(
        _stageB_body,
        out_shape=[jax.ShapeDtypeStruct((_F, _F), jnp.float32),
                   jax.ShapeDtypeStruct((_NP, _F), jnp.float32)],
        scratch_shapes=[pltpu.VMEM((_F, _F), jnp.float32),
                        pltpu.SMEM((_F,), jnp.int32),
                        pltpu.SMEM((_F,), jnp.float32)],
    )(AH, WtA, p_col, nrm, Wt, *gru_w)


def _stageC_body(AH_ref, W2t_ref, Ut_ref, Ub_ref, P_out, Q_out):
    AH = AH_ref[pl.ds(0, _NP), :]
    W2n = W2t_ref[...].T
    Y = jnp.dot(AH, W2n, preferred_element_type=jnp.float32)
    P_out[...] = jnp.dot(Y, Ut_ref[...], preferred_element_type=jnp.float32)
    Q_out[...] = jnp.dot(Y, Ub_ref[...], preferred_element_type=jnp.float32)


def _tc_stageC(AH, W2t, Ut, Ub):
    return pl.pallas_call(
        _stageC_body,
        out_shape=[jax.ShapeDtypeStruct((_NP, _F), jnp.float32),
                   jax.ShapeDtypeStruct((_NP, _F), jnp.float32)],
    )(AH, W2t, Ut, Ub)


_sc_mesh = plsc.VectorSubcoreMesh(core_axis_name="c", subcore_axis_name="s")


@functools.partial(
    pl.kernel, mesh=_sc_mesh,
    out_type=jax.ShapeDtypeStruct((_NPX, _F), jnp.float32),
    scratch_types=[
        pltpu.VMEM((_SS_WIN, _F), jnp.int32),
        pltpu.VMEM((_F, _F), jnp.float32),
        pltpu.VMEM((_SS_SPAN, _F), jnp.float32),
        pltpu.VMEM((_SS_SPAN // _F, _F), jnp.int32),
        pltpu.VMEM((1, 16), jnp.int32),
        pltpu.VMEM((_SS_WIN, _F), jnp.float32),
        pltpu.VMEM((1, _F), jnp.float32),
        pltpu.VMEM((_SS_WIN, _F), jnp.int32),
        pltpu.VMEM((16,), jnp.int32),
        pltpu.VMEM((1, _F), jnp.int32),
        pltpu.SemaphoreType.DMA,
    ])
def _csr_segsum(H_hbm, ss_hbm, sv_hbm, st_hbm, bounds_hbm, out_hbm,
                ssv, rowb, outb, idxb, bndv, svv, accv, stv, prevv, colv,
                sem):
    c = lax.axis_index("c")
    s = lax.axis_index("s")
    w = s * 2 + c

    pltpu.sync_copy(bounds_hbm.at[w], bndv)
    b = bndv[0, :]
    a0 = pl.multiple_of(b[0], 8)
    lb0 = b[1]
    lb1 = b[2]
    n0 = b[3]
    n1 = b[4]

    pltpu.sync_copy(ss_hbm.at[pl.ds(a0, _SS_WIN)], ssv)
    pltpu.sync_copy(sv_hbm.at[pl.ds(a0, _SS_WIN)], svv)
    pltpu.sync_copy(st_hbm.at[pl.ds(a0, _SS_WIN)], stv)

    zero16 = jnp.zeros((16,), jnp.float32)
    zidx16 = jnp.zeros((16,), jnp.int32)
    i16 = lax.broadcasted_iota(jnp.int32, (16,), 0)

    def zo(r, _):
        for cc in range(8):
            outb[r, pl.ds(cc * 16, 16)] = zero16
        return 0
    lax.fori_loop(0, _SS_SPAN, zo, 0)

    def flush(prevs):
        ro = prevs - n0
        for cc in range(8):
            outb[ro, pl.ds(cc * 16, 16)] = accv[0, pl.ds(cc * 16, 16)]

    def lane(l, prevs, bb, vv, tv):
        r = bb * 16 + l
        v16 = vv.at[zidx16 + l].get(mode="promise_in_bounds")
        stc16 = tv.at[zidx16 + l].get(mode="promise_in_bounds")
        prevv[...] = stc16 + zidx16
        stcs = prevv[...][0]
        change = stcs != prevs

        @pl.when(change & (prevs >= n0) & (prevs < n1))
        def _():
            flush(prevs)

        for cc in range(8):
            accv[0, pl.ds(cc * 16, 16)] = (
                rowb[r, pl.ds(cc * 16, 16)] * v16
                + jnp.where(change, zero16, accv[0, pl.ds(cc * 16, 16)]))
        return stcs

    def chunk(kc, prevs):
        pltpu.async_copy(H_hbm.at[ssv.at[kc]], rowb, sem).wait()
        for bb in range(8):
            vv = svv[kc, pl.ds(bb * 16, 16)]
            tv = stv[kc, pl.ds(bb * 16, 16)]
            prevs = lax.fori_loop(
                0, 16, functools.partial(lane, bb=bb, vv=vv, tv=tv), prevs)
        return prevs

    prevf = lax.fori_loop(lb0 // _F, (lb1 + _F - 1) // _F, chunk,
                          jnp.int32(-1))

    @pl.when((prevf >= n0) & (prevf < n1))
    def _():
        flush(prevf)

    l16 = lax.broadcasted_iota(jnp.int32, (16,), 0)
    nrow = n1 - n0
    for j in range(_SS_SPAN // _F):
        for g in range(8):
            rid = j * _F + g * 16 + l16
            idxb[j, pl.ds(g * 16, 16)] = jnp.where(
                rid < nrow, n0 + rid, _NP + (rid % _F))
    for j in range(_SS_SPAN // _F):
        pltpu.sync_copy(outb.at[pl.ds(j * _F, _F)], out_hbm.at[idxb.at[j]])


@functools.partial(
    pl.kernel, mesh=_sc_mesh,
    out_type=jax.ShapeDtypeStruct((_TEP, _F), jnp.float32),
    scratch_types=[
        pltpu.VMEM((_EG_CPW, _F), jnp.int32),
        pltpu.VMEM((_EG_CPW, _F), jnp.int32),
        pltpu.VMEM((_F, _F), jnp.float32),
        pltpu.VMEM((_F, _F), jnp.float32),
        pltpu.SemaphoreType.DMA,
        pltpu.SemaphoreType.DMA,
    ])
def _edge_gather(P_hbm, Q_hbm, sidx_hbm, tidx_hbm, out_hbm,
                 sv, tv, bufa, bufb, sema, semb):
    c = lax.axis_index("c")
    s = lax.axis_index("s")
    w = s * 2 + c
    base = w * _EG_CPW

    pltpu.sync_copy(sidx_hbm.at[w], sv)
    pltpu.sync_copy(tidx_hbm.at[w], tv)

    def chunk(i, _):
        cpa = pltpu.async_copy(P_hbm.at[sv.at[i]], bufa, sema)
        cpb = pltpu.async_copy(Q_hbm.at[tv.at[i]], bufb, semb)
        cpa.wait()
        cpb.wait()

        def addrow(r, _):
            for cc in range(8):
                bufa[r, pl.ds(cc * 16, 16)] = (
                    bufa[r, pl.ds(cc * 16, 16)]
                    + bufb[r, pl.ds(cc * 16, 16)])
            return 0

        lax.fori_loop(0, _F, addrow, 0)
        pltpu.sync_copy(bufa, out_hbm.at[pl.ds((base + i) * _F, _F)])
        return 0

    lax.fori_loop(0, _EG_CPW, chunk, 0)


def kernel(X, edges, A_values, W_init, W_init2, p,
           W_Z, U_Z, B_Z, W_R, U_R, B_R, W_H, U_H, B_H,
           p2, W_Z2, U_Z2, B_Z2, W_R2, U_R2, B_R2, W_H2, U_H2, B_H2, U):
    f32 = jnp.float32
    i32 = jnp.int32
    edges = edges.astype(i32)
    Xp = jnp.pad(X.astype(f32), ((0, 0), (0, _NP - _N), (0, 0)))

    src2 = edges[1].reshape(_T, _E)
    trg2 = edges[2].reshape(_T, _E)
    vals2 = A_values.astype(f32)

    st3, ss3, sv3 = lax.sort((trg2, src2, vals2), num_keys=1, is_stable=True)

    epadlen = _ECH * _F + (_SS_WIN + 8) * _F
    epad = epadlen - _E
    ssp = jnp.pad(ss3, ((0, 0), (0, epad))).reshape(_T, -1, _F)
    svp = jnp.pad(sv3, ((0, 0), (0, epad))).reshape(_T, -1, _F)
    stp = jnp.pad(st3, ((0, 0), (0, epad)),
                  constant_values=_NPX).reshape(_T, -1, _F)

    probe = jnp.arange(1, _NW, dtype=i32) * (_E // _NW)
    nodes_at = jnp.take_along_axis(st3, probe[None, :].repeat(_T, 0), axis=1)
    Bmid = jax.vmap(lambda a, v: jnp.searchsorted(a, v, side="left"))(
        st3, nodes_at).astype(i32)
    zc = jnp.zeros((_T, 1), i32)
    B = jnp.concatenate([zc, Bmid, jnp.full((_T, 1), _E, i32)], axis=1)
    nmid = nodes_at
    nvec = jnp.concatenate([zc, nmid, jnp.full((_T, 1), _NP, i32)], axis=1)
    a0 = (B[:, :_NW] // (8 * _F)) * 8
    bounds = jnp.stack([
        a0,
        B[:, :_NW] - a0 * _F,
        B[:, 1:] - a0 * _F,
        nvec[:, :_NW],
        nvec[:, 1:],
    ], axis=-1)
    bounds = jnp.pad(bounds, ((0, 0), (0, 0), (0, 11)))
    bounds = bounds.reshape(_T, _NW, 1, 16)

    gpad = _TEP - _TE
    srcf3d = jnp.pad(edges[0] * _NP + edges[1], (0, gpad)
                     ).reshape(_NW, _EG_CPW, _F)
    trgf3d = jnp.pad(edges[0] * _NP + edges[2], (0, gpad)
                     ).reshape(_NW, _EG_CPW, _F)

    gru1 = tuple(m.T for m in (W_Z, U_Z, B_Z, W_R, U_R, B_R, W_H, U_H, B_H))
    gru2 = tuple(m.T for m in (W_Z2, U_Z2, B_Z2, W_R2, U_R2, B_R2,
                               W_H2, U_H2, B_H2))
    p_col = p.reshape(_F, 1)
    p2_col = p2.reshape(_F, 1)
    nrm1 = jnp.linalg.norm(p).reshape(1, 1)
    nrm2 = jnp.linalg.norm(p2).reshape(1, 1)
    Ut = U[:_F]
    Ub = U[_F:]

    Wt = W_init.T
    W2t = W_init2.T
    Ps, Qs = [], []
    for t in range(_T):
        Wt = _tc_stageA(Xp[t], p_col, nrm1, Wt, gru1)
        AH1 = _csr_segsum(Xp[t], ssp[t], svp[t], stp[t], bounds[t])
        W2t, Xs1 = _tc_stageB(AH1, Wt, p2_col, nrm2, W2t, gru2)
        AH2 = _csr_segsum(Xs1, ssp[t], svp[t], stp[t], bounds[t])
        P_t, Q_t = _tc_stageC(AH2, W2t, Ut, Ub)
        Ps.append(P_t)
        Qs.append(Q_t)

    Pfull = jnp.concatenate(Ps, axis=0)
    Qfull = jnp.concatenate(Qs, axis=0)
    out = _edge_gather(Pfull, Qfull, srcf3d, trgf3d)
    return (out[:_TE], Wt.T, W2t.T)
